# Initial kernel scaffold; baseline (speedup 1.0000x reference)
import jax
import jax.numpy as jnp
from jax.experimental import pallas as pl
from jax.experimental.pallas import tpu as pltpu


def _match_body(q_ref, k_ref, s_ref, idx_ref):
    kn2 = jnp.zeros((1024,), jnp.float32)
    qn2 = jnp.zeros((1024,), jnp.float32)
    for dy in range(3):
        for dx in range(3):
            Ks = k_ref[:, dy:dy + 32, dx:dx + 32].reshape(256, 1024)
            Qs = q_ref[:, dy:dy + 32, dx:dx + 32].reshape(256, 1024)
            kn2 += jnp.sum(Ks * Ks, axis=0)
            qn2 += jnp.sum(Qs * Qs, axis=0)
    rk = 1.0 / jnp.maximum(jnp.sqrt(kn2), 1e-12)
    rq = 1.0 / jnp.maximum(jnp.sqrt(qn2), 1e-12)
    R = jnp.zeros((1024, 1024), jnp.float32)
    for dy in range(3):
        for dx in range(3):
            Ks = k_ref[:, dy:dy + 32, dx:dx + 32].reshape(256, 1024)
            Qs = q_ref[:, dy:dy + 32, dx:dx + 32].reshape(256, 1024)
            R += jax.lax.dot_general(
                Ks * rk[None, :], Qs * rq[None, :],
                (((0,), (0,)), ((), ())),
                preferred_element_type=jnp.float32)
    vmax = jnp.max(R, axis=0)
    rows = jax.lax.broadcasted_iota(jnp.int32, (1024, 1024), 0)
    hit = jnp.where(R == vmax[None, :], rows, 1024)
    idx = jnp.min(hit, axis=0)
    s_ref[...] = vmax
    idx_ref[...] = idx


def _match(qpad, kpad):
    return pl.pallas_call(
        _match_body,
        out_shape=(
            jax.ShapeDtypeStruct((1024,), jnp.float32),
            jax.ShapeDtypeStruct((1024,), jnp.int32),
        ),
    )(qpad, kpad)


def _transfer_body(idx_ref, t3_ref, t2_ref, t1_ref,
                   a3_ref, a2_ref, a1_ref):
    m = pl.program_id(0)

    @pl.when(m == 0)
    def _init():
        a3_ref[...] = jnp.zeros(a3_ref.shape, jnp.float32)
        a2_ref[...] = jnp.zeros(a2_ref.shape, jnp.float32)
        a1_ref[...] = jnp.zeros(a1_ref.shape, jnp.float32)

    r = idx_ref[m]
    ki = r // 32
    kj = r - ki * 32
    mi = m // 32
    mj = m - mi * 32
    a3_ref[pl.ds(mi, 3), pl.ds(mj, 3), :] += (
        t3_ref[pl.ds(ki, 3), pl.ds(kj, 3), :])
    a2_ref[pl.ds(2 * mi, 6), pl.ds(2 * mj, 6), :] += (
        t2_ref[pl.ds(2 * ki, 6), pl.ds(2 * kj, 6), :])
    a1_ref[pl.ds(4 * mi, 12), pl.ds(4 * mj, 12), :] += (
        t1_ref[pl.ds(4 * ki, 12), pl.ds(4 * kj, 12), :])


def _transfer(idx, t3, t2, t1):
    grid_spec = pltpu.PrefetchScalarGridSpec(
        num_scalar_prefetch=1,
        grid=(1024,),
        in_specs=[
            pl.BlockSpec((34, 34, 256), lambda m, s: (0, 0, 0)),
            pl.BlockSpec((68, 68, 128), lambda m, s: (0, 0, 0)),
            pl.BlockSpec((136, 136, 64), lambda m, s: (0, 0, 0)),
        ],
        out_specs=[
            pl.BlockSpec((34, 34, 256), lambda m, s: (0, 0, 0)),
            pl.BlockSpec((68, 68, 128), lambda m, s: (0, 0, 0)),
            pl.BlockSpec((136, 136, 64), lambda m, s: (0, 0, 0)),
        ],
    )
    return pl.pallas_call(
        _transfer_body,
        grid_spec=grid_spec,
        out_shape=(
            jax.ShapeDtypeStruct((34, 34, 256), jnp.float32),
            jax.ShapeDtypeStruct((68, 68, 128), jnp.float32),
            jax.ShapeDtypeStruct((136, 136, 64), jnp.float32),
        ),
    )(idx, t3, t2, t1)


def _count1d(h, sub, div, hi_add):
    lo = jnp.maximum(0, (h - sub + div - 1) // div)
    hi = jnp.minimum(31, (h + hi_add) // div)
    return (hi - lo + 1).astype(jnp.float32)


def _finalize_body(a3_ref, a2_ref, a1_ref, o3_ref, o2_ref, o1_ref):
    def scale(a_ref, o_ref, pad, n, sub, div, hi_add):
        hs = jax.lax.broadcasted_iota(jnp.int32, (n, n, 1), 0)
        ws = jax.lax.broadcasted_iota(jnp.int32, (n, n, 1), 1)
        cnt = _count1d(hs, sub, div, hi_add) * _count1d(ws, sub, div, hi_add)
        o_ref[...] = a_ref[pad:pad + n, pad:pad + n, :] / cnt

    scale(a3_ref, o3_ref, 1, 32, 1, 1, 1)
    scale(a2_ref, o2_ref, 2, 64, 3, 2, 2)
    scale(a1_ref, o1_ref, 4, 128, 7, 4, 4)


def _finalize(a3, a2, a1):
    return pl.pallas_call(
        _finalize_body,
        out_shape=(
            jax.ShapeDtypeStruct((32, 32, 256), jnp.float32),
            jax.ShapeDtypeStruct((64, 64, 128), jnp.float32),
            jax.ShapeDtypeStruct((128, 128, 64), jnp.float32),
        ),
    )(a3, a2, a1)


@jax.jit
def _run(query_lv3, key_lv3, value_lv1, value_lv2, value_lv3):
    qpad = jnp.pad(query_lv3[0], ((0, 0), (1, 1), (1, 1)))
    kpad = jnp.pad(key_lv3[0], ((0, 0), (1, 1), (1, 1)))
    s, idx = _match(qpad, kpad)

    t3 = jnp.pad(value_lv3[0], ((0, 0), (1, 1), (1, 1))).transpose(1, 2, 0)
    t2 = jnp.pad(value_lv2[0], ((0, 0), (2, 2), (2, 2))).transpose(1, 2, 0)
    t1 = jnp.pad(value_lv1[0], ((0, 0), (4, 4), (4, 4))).transpose(1, 2, 0)
    a3, a2, a1 = _transfer(idx, t3, t2, t1)
    o3, o2, o1 = _finalize(a3, a2, a1)

    S = s.reshape(1, 1, 32, 32)
    T_lv3 = o3.transpose(2, 0, 1)[None]
    T_lv2 = o2.transpose(2, 0, 1)[None]
    T_lv1 = o1.transpose(2, 0, 1)[None]
    return S, T_lv3, T_lv2, T_lv1


def kernel(query_lv3, key_lv3, value_lv1, value_lv2, value_lv3,
           cl_ref, dr_img):
    return _run(query_lv3, key_lv3, value_lv1, value_lv2, value_lv3)


# speedup vs baseline: 20.2317x; 20.2317x over previous
import jax
import jax.numpy as jnp
from jax.experimental import pallas as pl
from jax.experimental.pallas import tpu as pltpu


def _match_body(q_ref, k_ref, s_ref, idx_ref):
    shifts = [(dy, dx) for dy in range(3) for dx in range(3)]
    qn2 = jnp.zeros((1024,), jnp.float32)
    for dy, dx in shifts:
        Qs = q_ref[:, dy:dy + 32, dx:dx + 32].reshape(256, 1024)
        qn2 += jnp.sum(Qs * Qs, axis=0)
    rq = 1.0 / jnp.maximum(jnp.sqrt(qn2), 1e-12)
    Qsc = [q_ref[:, dy:dy + 32, dx:dx + 32].reshape(256, 1024) * rq[None, :]
           for dy, dx in shifts]

    vals = jnp.full((1024,), -jnp.inf, jnp.float32)
    idx = jnp.zeros((1024,), jnp.int32)
    rows0 = jax.lax.broadcasted_iota(jnp.int32, (128, 1024), 0)
    for b in range(8):
        kn2 = jnp.zeros((128,), jnp.float32)
        Kb = []
        for dy, dx in shifts:
            Ks = k_ref[:, 4 * b + dy:4 * b + dy + 4,
                       dx:dx + 32].reshape(256, 128)
            kn2 += jnp.sum(Ks * Ks, axis=0)
            Kb.append(Ks)
        rk = 1.0 / jnp.maximum(jnp.sqrt(kn2), 1e-12)
        Rb = jnp.zeros((128, 1024), jnp.float32)
        for s in range(9):
            Rb += jax.lax.dot_general(
                Kb[s] * rk[None, :], Qsc[s],
                (((0,), (0,)), ((), ())),
                preferred_element_type=jnp.float32)
        bmax = jnp.max(Rb, axis=0)
        hit = jnp.where(Rb == bmax[None, :], rows0 + 128 * b, 2048)
        bidx = jnp.min(hit, axis=0)
        better = bmax > vals
        vals = jnp.where(better, bmax, vals)
        idx = jnp.where(better, bidx, idx)
    s_ref[...] = vals
    idx_ref[...] = idx


def _match(qpad, kpad):
    return pl.pallas_call(
        _match_body,
        out_shape=(
            jax.ShapeDtypeStruct((1024,), jnp.float32),
            jax.ShapeDtypeStruct((1024,), jnp.int32),
        ),
    )(qpad, kpad)


def _transfer_body(idx_ref, t3_ref, t2_ref, t1_ref,
                   a3_ref, a2_ref, a1_ref):
    m = pl.program_id(0)

    @pl.when(m == 0)
    def _init():
        a3_ref[...] = jnp.zeros(a3_ref.shape, jnp.float32)
        a2_ref[...] = jnp.zeros(a2_ref.shape, jnp.float32)
        a1_ref[...] = jnp.zeros(a1_ref.shape, jnp.float32)

    r = idx_ref[m]
    ki = r // 32
    kj = r - ki * 32
    mi = m // 32
    mj = m - mi * 32
    for dy in range(3):
        a3_ref[pl.ds((mi + dy) * 34 + mj, 3), :, :] += (
            t3_ref[pl.ds((ki + dy) * 34 + kj, 3), :, :])
    for dy in range(6):
        a2_ref[pl.ds((2 * mi + dy) * 68 + 2 * mj, 6), :, :] += (
            t2_ref[pl.ds((2 * ki + dy) * 68 + 2 * kj, 6), :, :])
    for dy in range(12):
        a1_ref[pl.ds((4 * mi + dy) * 136 + 4 * mj, 12), :, :] += (
            t1_ref[pl.ds((4 * ki + dy) * 136 + 4 * kj, 12), :, :])


def _transfer(idx, t3, t2, t1):
    grid_spec = pltpu.PrefetchScalarGridSpec(
        num_scalar_prefetch=1,
        grid=(1024,),
        in_specs=[
            pl.BlockSpec((34 * 34, 1, 256), lambda m, s: (0, 0, 0)),
            pl.BlockSpec((68 * 68, 1, 128), lambda m, s: (0, 0, 0)),
            pl.BlockSpec((136 * 136, 1, 64), lambda m, s: (0, 0, 0)),
        ],
        out_specs=[
            pl.BlockSpec((34 * 34, 1, 256), lambda m, s: (0, 0, 0)),
            pl.BlockSpec((68 * 68, 1, 128), lambda m, s: (0, 0, 0)),
            pl.BlockSpec((136 * 136, 1, 64), lambda m, s: (0, 0, 0)),
        ],
    )
    return pl.pallas_call(
        _transfer_body,
        grid_spec=grid_spec,
        out_shape=(
            jax.ShapeDtypeStruct((34 * 34, 1, 256), jnp.float32),
            jax.ShapeDtypeStruct((68 * 68, 1, 128), jnp.float32),
            jax.ShapeDtypeStruct((136 * 136, 1, 64), jnp.float32),
        ),
    )(idx, t3, t2, t1)


def _count1d(h, sub, div, hi_add):
    lo = jnp.maximum(0, (h - sub + div - 1) // div)
    hi = jnp.minimum(31, (h + hi_add) // div)
    return (hi - lo + 1).astype(jnp.float32)


def _finalize_body(a3_ref, a2_ref, a1_ref, o3_ref, o2_ref, o1_ref):
    def scale(a_ref, o_ref, pad, n, sub, div, hi_add):
        hs = jax.lax.broadcasted_iota(jnp.int32, (n, n, 1), 0)
        ws = jax.lax.broadcasted_iota(jnp.int32, (n, n, 1), 1)
        cnt = _count1d(hs, sub, div, hi_add) * _count1d(ws, sub, div, hi_add)
        o_ref[...] = a_ref[pad:pad + n, pad:pad + n, :] / cnt

    scale(a3_ref, o3_ref, 1, 32, 1, 1, 1)
    scale(a2_ref, o2_ref, 2, 64, 3, 2, 2)
    scale(a1_ref, o1_ref, 4, 128, 7, 4, 4)


def _finalize(a3, a2, a1):
    return pl.pallas_call(
        _finalize_body,
        out_shape=(
            jax.ShapeDtypeStruct((32, 32, 256), jnp.float32),
            jax.ShapeDtypeStruct((64, 64, 128), jnp.float32),
            jax.ShapeDtypeStruct((128, 128, 64), jnp.float32),
        ),
    )(a3, a2, a1)


@jax.jit
def _run(query_lv3, key_lv3, value_lv1, value_lv2, value_lv3):
    qpad = jnp.pad(query_lv3[0], ((0, 0), (1, 1), (1, 1)))
    kpad = jnp.pad(key_lv3[0], ((0, 0), (1, 1), (1, 1)))
    s, idx = _match(qpad, kpad)

    t3 = jnp.pad(value_lv3[0], ((0, 0), (1, 1), (1, 1))).transpose(1, 2, 0)
    t2 = jnp.pad(value_lv2[0], ((0, 0), (2, 2), (2, 2))).transpose(1, 2, 0)
    t1 = jnp.pad(value_lv1[0], ((0, 0), (4, 4), (4, 4))).transpose(1, 2, 0)
    a3, a2, a1 = _transfer(idx,
                           t3.reshape(34 * 34, 1, 256),
                           t2.reshape(68 * 68, 1, 128),
                           t1.reshape(136 * 136, 1, 64))
    o3, o2, o1 = _finalize(a3.reshape(34, 34, 256),
                           a2.reshape(68, 68, 128),
                           a1.reshape(136, 136, 64))

    S = s.reshape(1, 1, 32, 32)
    T_lv3 = o3.transpose(2, 0, 1)[None]
    T_lv2 = o2.transpose(2, 0, 1)[None]
    T_lv1 = o1.transpose(2, 0, 1)[None]
    return S, T_lv3, T_lv2, T_lv1


def kernel(query_lv3, key_lv3, value_lv1, value_lv2, value_lv3,
           cl_ref, dr_img):
    return _run(query_lv3, key_lv3, value_lv1, value_lv2, value_lv3)
